# Initial kernel scaffold; baseline (speedup 1.0000x reference)
import jax
import jax.numpy as jnp
from jax.experimental import pallas as pl


def kernel(x, edge_index, batch_idx, params):
    raise NotImplementedError("write your pallas kernel here")


# speedup vs baseline: 2.9212x; 2.9212x over previous
import functools

import jax
import jax.numpy as jnp
from jax import lax
from jax.experimental import pallas as pl
from jax.experimental.pallas import tpu as pltpu
from jax.experimental.pallas import tpu_sc as plsc

_N_NODES = 10000
_N_EDGES = 320000
_D = 128
_N_GRAPHS = 64
_BN_EPS = 1e-5

_NC = 2
_NS = 16
_NW = _NC * _NS


@functools.lru_cache(maxsize=None)
def _make_seg_sum(n_edges_pad, chunk, acc_rows):
    n_chunks = n_edges_pad // (_NW * chunk)
    assert n_chunks * _NW * chunk == n_edges_pad
    per_worker = n_chunks * chunk
    rows_per_tile = acc_rows // _NS
    assert rows_per_tile * _NS == acc_rows
    zr = 1
    for cand in range(1, 65):
        if rows_per_tile % cand == 0:
            zr = cand
    n_zero_copies = rows_per_tile // zr

    mesh = plsc.VectorSubcoreMesh(core_axis_name="c", subcore_axis_name="s",
                                  num_cores=_NC, num_subcores=_NS)

    @functools.partial(
        pl.kernel,
        out_type=jax.ShapeDtypeStruct((_NC, acc_rows, _D), jnp.float32),
        mesh=mesh,
        scratch_types=[
            pltpu.VMEM((chunk,), jnp.int32),
            pltpu.VMEM((chunk,), jnp.int32),
            pltpu.VMEM((chunk, _D), jnp.float32),
            pltpu.VMEM((zr, _D), jnp.float32),
            pltpu.VMEM_SHARED((acc_rows, _D), jnp.float32),
            pltpu.SemaphoreType.DMA,
        ],
    )
    def seg_sum(table_hbm, src_hbm, dst_hbm, out_hbm,
                src_v, dst_v, rows_v, zero_v, acc_sh, sem):
        c = lax.axis_index("c")
        s = lax.axis_index("s")
        wid = s * _NC + c

        zeros16 = jnp.zeros((16,), jnp.float32)

        def zrow(r, _):
            for col in range(_D // 16):
                zero_v[r, pl.ds(col * 16, 16)] = zeros16
            return ()

        lax.fori_loop(0, zr, zrow, ())
        row0 = s * rows_per_tile
        for t in range(n_zero_copies):
            pltpu.sync_copy(zero_v, acc_sh.at[pl.ds(row0 + t * zr, zr)])
        plsc.subcore_barrier()

        base0 = wid * per_worker

        def body(j, _):
            base = base0 + j * chunk
            pltpu.sync_copy(src_hbm.at[pl.ds(base, chunk)], src_v)
            pltpu.sync_copy(dst_hbm.at[pl.ds(base, chunk)], dst_v)
            pltpu.async_copy(table_hbm.at[src_v], rows_v, sem).wait()
            pltpu.sync_copy(rows_v, acc_sh.at[dst_v], add=True)
            return ()

        lax.fori_loop(0, n_chunks, body, ())
        plsc.subcore_barrier()

        pltpu.sync_copy(acc_sh.at[pl.ds(row0, rows_per_tile)],
                        out_hbm.at[c].at[pl.ds(row0, rows_per_tile)])

    return seg_sum


_ACC_ROWS = 10240
_EDGE_CHUNK = 128
_E_PAD = 323584
_POOL_ACC_ROWS = 128
_POOL_CHUNK = 64
_POOL_E_PAD = 10240


_BR = 200
_NB = _N_NODES // _BR


def _mlp_block(h_ref, agg_ref, w1_ref, b1_ref, w2_ref, b2_ref,
               z_ref, s_ref, ss_ref):
    i = pl.program_id(0)
    x = h_ref[...] + agg_ref[0] + agg_ref[1]
    t = jnp.maximum(
        jnp.dot(x, w1_ref[...], preferred_element_type=jnp.float32)
        + b1_ref[...], 0.0)
    z = jnp.maximum(
        jnp.dot(t, w2_ref[...], preferred_element_type=jnp.float32)
        + b2_ref[...], 0.0)
    z_ref[...] = z

    @pl.when(i == 0)
    def _():
        s_ref[...] = jnp.zeros_like(s_ref)
        ss_ref[...] = jnp.zeros_like(ss_ref)

    zr = z.reshape(_BR // 8, 8, _D)
    s_ref[...] += jnp.sum(zr, axis=0)
    ss_ref[...] += jnp.sum(zr * zr, axis=0)


def _mlp_call(h, aggp, w1, b1, w2, b2):
    return pl.pallas_call(
        _mlp_block,
        grid=(_NB,),
        in_specs=[
            pl.BlockSpec((_BR, _D), lambda i: (i, 0)),
            pl.BlockSpec((_NC, _BR, _D), lambda i: (0, i, 0)),
            pl.BlockSpec((_D, _D), lambda i: (0, 0)),
            pl.BlockSpec((1, _D), lambda i: (0, 0)),
            pl.BlockSpec((_D, _D), lambda i: (0, 0)),
            pl.BlockSpec((1, _D), lambda i: (0, 0)),
        ],
        out_specs=[
            pl.BlockSpec((_BR, _D), lambda i: (i, 0)),
            pl.BlockSpec((8, _D), lambda i: (0, 0)),
            pl.BlockSpec((8, _D), lambda i: (0, 0)),
        ],
        out_shape=[
            jax.ShapeDtypeStruct((_N_NODES, _D), jnp.float32),
            jax.ShapeDtypeStruct((8, _D), jnp.float32),
            jax.ShapeDtypeStruct((8, _D), jnp.float32),
        ],
    )(h, aggp, w1, b1, w2, b2)


def _norm_block(z_ref, s_ref, ss_ref, g_ref, b_ref, o_ref):
    inv_n = 1.0 / _N_NODES
    mean = jnp.sum(s_ref[...], axis=0, keepdims=True) * inv_n
    var = jnp.sum(ss_ref[...], axis=0, keepdims=True) * inv_n - mean * mean
    scale = lax.rsqrt(var + _BN_EPS) * g_ref[...]
    o_ref[...] = (z_ref[...] - mean) * scale + b_ref[...]


def _norm_call(z, s, ss, gamma, beta):
    return pl.pallas_call(
        _norm_block,
        grid=(_NB,),
        in_specs=[
            pl.BlockSpec((_BR, _D), lambda i: (i, 0)),
            pl.BlockSpec((8, _D), lambda i: (0, 0)),
            pl.BlockSpec((8, _D), lambda i: (0, 0)),
            pl.BlockSpec((1, _D), lambda i: (0, 0)),
            pl.BlockSpec((1, _D), lambda i: (0, 0)),
        ],
        out_specs=pl.BlockSpec((_BR, _D), lambda i: (i, 0)),
        out_shape=jax.ShapeDtypeStruct((_N_NODES, _D), jnp.float32),
    )(z, s, ss, gamma, beta)


def _final_block(p_ref, wp_ref, bp_ref, o_ref):
    p = p_ref[0] + p_ref[1]
    o_ref[...] = jnp.maximum(
        jnp.dot(p, wp_ref[...], preferred_element_type=jnp.float32)
        + bp_ref[...], 0.0)


def _final_call(poolp, wp, bp):
    return pl.pallas_call(
        _final_block,
        grid=(1,),
        in_specs=[
            pl.BlockSpec((_NC, _N_GRAPHS, _D), lambda i: (0, 0, 0)),
            pl.BlockSpec((_D, _D), lambda i: (0, 0)),
            pl.BlockSpec((1, _D), lambda i: (0, 0)),
        ],
        out_specs=pl.BlockSpec((_N_GRAPHS, _D), lambda i: (0, 0)),
        out_shape=jax.ShapeDtypeStruct((_N_GRAPHS, _D), jnp.float32),
    )(poolp, wp, bp)


def kernel(x, edge_index, batch_idx, params):
    src = edge_index[0].astype(jnp.int32)
    dst = edge_index[1].astype(jnp.int32)
    e_pad = _E_PAD - _N_EDGES
    src_p = jnp.concatenate([src, jnp.zeros((e_pad,), jnp.int32)])
    dst_p = jnp.concatenate([dst, jnp.full((e_pad,), _N_NODES, jnp.int32)])

    n_pad = _POOL_E_PAD - _N_NODES
    src_pool = jnp.concatenate(
        [jnp.arange(_N_NODES, dtype=jnp.int32),
         jnp.zeros((n_pad,), jnp.int32)])
    dst_pool = jnp.concatenate(
        [batch_idx.astype(jnp.int32),
         jnp.full((n_pad,), _N_GRAPHS, jnp.int32)])

    seg_sum_edges = _make_seg_sum(_E_PAD, _EDGE_CHUNK, _ACC_ROWS)
    seg_sum_pool = _make_seg_sum(_POOL_E_PAD, _POOL_CHUNK, _POOL_ACC_ROWS)

    h = x
    for layer in params['layers']:
        aggp = seg_sum_edges(h, src_p, dst_p)
        z, s, ss = _mlp_call(
            h, aggp,
            layer['W1'], layer['b1'].reshape(1, _D),
            layer['W2'], layer['b2'].reshape(1, _D))
        h = _norm_call(z, s, ss,
                       layer['gamma'].reshape(1, _D),
                       layer['beta'].reshape(1, _D))

    poolp = seg_sum_pool(h, src_pool, dst_pool)
    return _final_call(poolp[:, :_N_GRAPHS, :], params['Wp'],
                       params['bp'].reshape(1, _D))
